# baseline (speedup 1.0000x reference)
import functools

import jax
import jax.numpy as jnp
from jax import lax
from jax.experimental import pallas as pl
from jax.experimental.pallas import tpu as pltpu
from jax.experimental.pallas import tpu_sc as plsc

N_NODES = 10000
D_IN = 128
D_OUT = 128

NCORES = 2
NSUB = 16
NWORKERS = NCORES * NSUB
CHUNK = 128
N_PAD_NODES = 10112
PAD_ROWS_PER_TILE = N_PAD_NODES // NSUB
HGRID = 80
EB = 32
BLK = 1000
BLK_L = PAD_ROWS_PER_TILE


def _sc_mesh():
  return plsc.VectorSubcoreMesh(core_axis_name="c", subcore_axis_name="s",
                                num_cores=1)


def _hist_body(d_ref, dinv_ref, acc_ref):
  i = pl.program_id(0)
  n = pl.num_programs(0)

  @pl.when(i == 0)
  def _():
    acc_ref[...] = jnp.zeros_like(acc_ref)

  d = d_ref[...]
  hi = lax.shift_right_logical(d, 7)
  lo = lax.bitwise_and(d, 127)
  ia = lax.broadcasted_iota(jnp.int32, (HGRID, 1), 0)
  ib = lax.broadcasted_iota(jnp.int32, (128, 1), 0)
  acc = jnp.zeros((HGRID, 128), jnp.float32)
  for r in range(EB):
    oh_hi = (hi[r:r + 1, :] == ia).astype(jnp.float32)
    oh_lo = (lo[r:r + 1, :] == ib).astype(jnp.float32)
    acc = acc + lax.dot_general(oh_hi, oh_lo, (((1,), (1,)), ((), ())),
                                preferred_element_type=jnp.float32)
  acc_ref[...] += acc

  @pl.when(i == n - 1)
  def _():
    dinv_ref[...] = lax.rsqrt(acc_ref[...] + 1.0)


def _hist_call(dst2d):
  n_rows = dst2d.shape[0]
  return pl.pallas_call(
      _hist_body,
      grid=(n_rows // EB,),
      in_specs=[pl.BlockSpec((EB, 128), lambda i: (i, 0))],
      out_specs=pl.BlockSpec((HGRID, 128), lambda i: (0, 0)),
      out_shape=jax.ShapeDtypeStruct((HGRID, 128), jnp.float32),
      scratch_shapes=[pltpu.VMEM((HGRID, 128), jnp.float32)],
  )(dst2d)


@functools.lru_cache(maxsize=None)
def _msg_call(cpt):
  @functools.partial(
      pl.kernel,
      out_type=jax.ShapeDtypeStruct((1, N_PAD_NODES, D_OUT), jnp.float32),
      mesh=_sc_mesh(),
      scratch_types=[
          pltpu.VMEM(((2 * cpt) // 5, CHUNK), jnp.int32),
          pltpu.VMEM(((2 * cpt) // 5, CHUNK), jnp.int32),
          pltpu.VMEM((CHUNK, D_OUT), jnp.float32),
          pltpu.VMEM((CHUNK, D_OUT), jnp.float32),
          pltpu.VMEM_SHARED((N_PAD_NODES, D_OUT), jnp.float32),
          pltpu.SemaphoreType.DMA,
          pltpu.SemaphoreType.DMA,
      ],
  )
  def msg_kernel(g_hbm, src_hbm, dst_hbm, out_hbm, src_v, dst_v, rows_v,
                 rows_b, acc_sh, sem, sem_b):
    c = lax.axis_index("c")
    s = lax.axis_index("s")
    wid = s * NCORES + c

    row0 = s * PAD_ROWS_PER_TILE
    for off in range(0, PAD_ROWS_PER_TILE, CHUNK):
      n = min(CHUNK, PAD_ROWS_PER_TILE - off)
      pltpu.sync_copy(g_hbm.at[pl.ds(row0 + off, n)], rows_v.at[pl.ds(0, n)])
      pltpu.sync_copy(rows_v.at[pl.ds(0, n)], acc_sh.at[pl.ds(row0 + off, n)])

    plsc.subcore_barrier()

    ph = (2 * cpt) // 5

    def edge_phase(pbase):
      pltpu.sync_copy(src_hbm.at[pl.ds(pbase, ph)], src_v)
      pltpu.sync_copy(dst_hbm.at[pl.ds(pbase, ph)], dst_v)
      pltpu.async_copy(g_hbm.at[src_v.at[0]], rows_v, sem)

      def pair(t, _):
        c0 = 2 * t
        pltpu.make_async_copy(g_hbm.at[src_v.at[c0]], rows_v, sem).wait()
        pltpu.async_copy(g_hbm.at[src_v.at[c0 + 1]], rows_b, sem_b)
        pltpu.sync_copy(rows_v, acc_sh.at[dst_v.at[c0]], add=True)
        pltpu.make_async_copy(g_hbm.at[src_v.at[c0 + 1]], rows_b,
                              sem_b).wait()
        pltpu.async_copy(g_hbm.at[src_v.at[c0 + 2]], rows_v, sem)
        pltpu.sync_copy(rows_b, acc_sh.at[dst_v.at[c0 + 1]], add=True)
        return 0

      lax.fori_loop(0, ph // 2 - 1, pair, 0)
      pltpu.make_async_copy(g_hbm.at[src_v.at[ph - 2]], rows_v, sem).wait()
      pltpu.async_copy(g_hbm.at[src_v.at[ph - 1]], rows_b, sem_b)
      pltpu.sync_copy(rows_v, acc_sh.at[dst_v.at[ph - 2]], add=True)
      pltpu.make_async_copy(g_hbm.at[src_v.at[ph - 1]], rows_b, sem_b).wait()
      pltpu.sync_copy(rows_b, acc_sh.at[dst_v.at[ph - 1]], add=True)

    for phase in range(5):
      edge_phase(s * 5 * ph + phase * ph)

    plsc.subcore_barrier()

    for off in range(0, PAD_ROWS_PER_TILE, CHUNK):
      n = min(CHUNK, PAD_ROWS_PER_TILE - off)
      pltpu.sync_copy(acc_sh.at[pl.ds(row0 + off, n)], rows_v.at[pl.ds(0, n)])
      pltpu.sync_copy(rows_v.at[pl.ds(0, n)],
                      out_hbm.at[c, pl.ds(row0 + off, n)])

  return msg_kernel


def _lin_body(x_ref, w_ref, dinv_ref, g_ref):
  h = jnp.dot(x_ref[...], w_ref[...], preferred_element_type=jnp.float32)
  g_ref[...] = h * dinv_ref[...]


def _linear_call(xp, W, dinv_col):
  return pl.pallas_call(
      _lin_body,
      grid=(N_PAD_NODES // BLK_L,),
      in_specs=[
          pl.BlockSpec((BLK_L, D_IN), lambda i: (i, 0)),
          pl.BlockSpec((D_IN, D_OUT), lambda i: (0, 0)),
          pl.BlockSpec((BLK_L, 1), lambda i: (i, 0)),
      ],
      out_specs=pl.BlockSpec((BLK_L, D_OUT), lambda i: (i, 0)),
      out_shape=jax.ShapeDtypeStruct((N_PAD_NODES, D_OUT), jnp.float32),
  )(xp, W, dinv_col)


def _out_body(p_ref, dinv_ref, b_ref, o_ref):
  z = p_ref[0] * dinv_ref[...] + b_ref[...]
  m = jnp.max(z, axis=1, keepdims=True)
  zs = z - m
  o_ref[...] = zs - jnp.log(jnp.sum(jnp.exp(zs), axis=1, keepdims=True))


def _final_call(acc_part, dinv_col, b2):
  return pl.pallas_call(
      _out_body,
      grid=(N_NODES // BLK,),
      in_specs=[
          pl.BlockSpec((1, BLK, D_OUT), lambda i: (0, i, 0)),
          pl.BlockSpec((BLK, 1), lambda i: (i, 0)),
          pl.BlockSpec((1, D_OUT), lambda i: (0, 0)),
      ],
      out_specs=pl.BlockSpec((BLK, D_OUT), lambda i: (i, 0)),
      out_shape=jax.ShapeDtypeStruct((N_NODES, D_OUT), jnp.float32),
  )(acc_part, dinv_col, b2)


def kernel(x, edge_index, W, b):
  n_edges = edge_index.shape[1]
  per_round = CHUNK * NWORKERS
  cpt = -(-n_edges // per_round)
  cpt = -(-cpt // 8) * 8
  e_pad = cpt * per_round
  pad = e_pad - n_edges

  src = jnp.concatenate(
      [edge_index[0], jnp.zeros((pad,), jnp.int32)]).reshape(-1, CHUNK)
  pad_dst = N_NODES + (jnp.arange(pad, dtype=jnp.int32)
                       % (N_PAD_NODES - N_NODES))
  dst = jnp.concatenate([edge_index[1], pad_dst]).reshape(-1, CHUNK)

  dinv2d = _hist_call(dst)
  dinv_col = dinv2d.reshape(HGRID * 128, 1)[:N_PAD_NODES]
  xp = jnp.concatenate(
      [x, jnp.zeros((N_PAD_NODES - N_NODES, D_IN), jnp.float32)])
  g = _linear_call(xp, W, dinv_col)
  acc_part = _msg_call(cpt)(g, src, dst)
  return _final_call(acc_part, dinv_col[:N_NODES], b.reshape(1, D_OUT))


# speedup vs baseline: 1.1469x; 1.1469x over previous
import functools

import jax
import jax.numpy as jnp
from jax import lax
from jax.experimental import pallas as pl
from jax.experimental.pallas import tpu as pltpu
from jax.experimental.pallas import tpu_sc as plsc

N_NODES = 10000
D_IN = 128
D_OUT = 128

NCORES = 2
NSUB = 16
NWORKERS = NCORES * NSUB
CHUNK = 128
N_PAD_NODES = 10112
PAD_ROWS_PER_TILE = N_PAD_NODES // NSUB
HGRID = 80
EB = 32
BLK = 1000
BLK_L = PAD_ROWS_PER_TILE


def _sc_mesh():
  return plsc.VectorSubcoreMesh(core_axis_name="c", subcore_axis_name="s")


def _hist_body(d_ref, dinv_ref, acc_ref):
  i = pl.program_id(0)
  n = pl.num_programs(0)

  @pl.when(i == 0)
  def _():
    acc_ref[...] = jnp.zeros_like(acc_ref)

  d = d_ref[...]
  hi = lax.shift_right_logical(d, 7)
  lo = lax.bitwise_and(d, 127)
  ia = lax.broadcasted_iota(jnp.int32, (HGRID, 1), 0)
  ib = lax.broadcasted_iota(jnp.int32, (128, 1), 0)
  acc = jnp.zeros((HGRID, 128), jnp.float32)
  for r in range(EB):
    oh_hi = (hi[r:r + 1, :] == ia).astype(jnp.float32)
    oh_lo = (lo[r:r + 1, :] == ib).astype(jnp.float32)
    acc = acc + lax.dot_general(oh_hi, oh_lo, (((1,), (1,)), ((), ())),
                                preferred_element_type=jnp.float32)
  acc_ref[...] += acc

  @pl.when(i == n - 1)
  def _():
    dinv_ref[...] = lax.rsqrt(acc_ref[...] + 1.0)


def _hist_call(dst2d):
  n_rows = dst2d.shape[0]
  return pl.pallas_call(
      _hist_body,
      grid=(n_rows // EB,),
      in_specs=[pl.BlockSpec((EB, 128), lambda i: (i, 0))],
      out_specs=pl.BlockSpec((HGRID, 128), lambda i: (0, 0)),
      out_shape=jax.ShapeDtypeStruct((HGRID, 128), jnp.float32),
      scratch_shapes=[pltpu.VMEM((HGRID, 128), jnp.float32)],
  )(dst2d)


@functools.lru_cache(maxsize=None)
def _msg_call(cpt):
  @functools.partial(
      pl.kernel,
      out_type=jax.ShapeDtypeStruct((NCORES, N_PAD_NODES, D_OUT), jnp.float32),
      mesh=_sc_mesh(),
      scratch_types=[
          pltpu.VMEM(((2 * cpt) // 5, CHUNK), jnp.int32),
          pltpu.VMEM(((2 * cpt) // 5, CHUNK), jnp.int32),
          pltpu.VMEM((CHUNK, D_OUT), jnp.float32),
          pltpu.VMEM((CHUNK, D_OUT), jnp.float32),
          pltpu.VMEM_SHARED((N_PAD_NODES, D_OUT), jnp.float32),
          pltpu.SemaphoreType.DMA,
          pltpu.SemaphoreType.DMA,
      ],
  )
  def msg_kernel(g_hbm, src_hbm, dst_hbm, out_hbm, src_v, dst_v, rows_v,
                 rows_b, acc_sh, sem, sem_b):
    c = lax.axis_index("c")
    s = lax.axis_index("s")
    wid = s * NCORES + c

    row0 = s * PAD_ROWS_PER_TILE

    @pl.when(c == 0)
    def _():
      for off in range(0, PAD_ROWS_PER_TILE, CHUNK):
        n = min(CHUNK, PAD_ROWS_PER_TILE - off)
        pltpu.sync_copy(g_hbm.at[pl.ds(row0 + off, n)], rows_v.at[pl.ds(0, n)])
        pltpu.sync_copy(rows_v.at[pl.ds(0, n)],
                        acc_sh.at[pl.ds(row0 + off, n)])

    @pl.when(c != 0)
    def _():
      def fillz(i, _):
        for k in range(8):
          rows_v[i, pl.ds(k * 16, 16)] = jnp.zeros((16,), jnp.float32)
        return 0

      lax.fori_loop(0, CHUNK, fillz, 0)
      for off in range(0, PAD_ROWS_PER_TILE, CHUNK):
        n = min(CHUNK, PAD_ROWS_PER_TILE - off)
        pltpu.sync_copy(rows_v.at[pl.ds(0, n)],
                        acc_sh.at[pl.ds(row0 + off, n)])

    plsc.subcore_barrier()

    ph = (2 * cpt) // 5

    def edge_phase(pbase):
      pltpu.sync_copy(src_hbm.at[pl.ds(pbase, ph)], src_v)
      pltpu.sync_copy(dst_hbm.at[pl.ds(pbase, ph)], dst_v)
      pltpu.async_copy(g_hbm.at[src_v.at[0]], rows_v, sem)

      def pair(t, _):
        c0 = 2 * t
        pltpu.make_async_copy(g_hbm.at[src_v.at[c0]], rows_v, sem).wait()
        pltpu.async_copy(g_hbm.at[src_v.at[c0 + 1]], rows_b, sem_b)
        pltpu.sync_copy(rows_v, acc_sh.at[dst_v.at[c0]], add=True)
        pltpu.make_async_copy(g_hbm.at[src_v.at[c0 + 1]], rows_b,
                              sem_b).wait()
        pltpu.async_copy(g_hbm.at[src_v.at[c0 + 2]], rows_v, sem)
        pltpu.sync_copy(rows_b, acc_sh.at[dst_v.at[c0 + 1]], add=True)
        return 0

      lax.fori_loop(0, ph // 2 - 1, pair, 0)
      pltpu.make_async_copy(g_hbm.at[src_v.at[ph - 2]], rows_v, sem).wait()
      pltpu.async_copy(g_hbm.at[src_v.at[ph - 1]], rows_b, sem_b)
      pltpu.sync_copy(rows_v, acc_sh.at[dst_v.at[ph - 2]], add=True)
      pltpu.make_async_copy(g_hbm.at[src_v.at[ph - 1]], rows_b, sem_b).wait()
      pltpu.sync_copy(rows_b, acc_sh.at[dst_v.at[ph - 1]], add=True)

    @pl.when(c == 0)
    def _():
      for phase in range(3):
        edge_phase(s * 3 * ph + phase * ph)

    @pl.when(c != 0)
    def _():
      for phase in range(2):
        edge_phase(NSUB * 3 * ph + s * 2 * ph + phase * ph)

    plsc.subcore_barrier()

    for off in range(0, PAD_ROWS_PER_TILE, CHUNK):
      n = min(CHUNK, PAD_ROWS_PER_TILE - off)
      pltpu.sync_copy(acc_sh.at[pl.ds(row0 + off, n)], rows_v.at[pl.ds(0, n)])
      pltpu.sync_copy(rows_v.at[pl.ds(0, n)],
                      out_hbm.at[c, pl.ds(row0 + off, n)])

  return msg_kernel


def _lin_body(x_ref, w_ref, dinv_ref, g_ref):
  h = jnp.dot(x_ref[...], w_ref[...], preferred_element_type=jnp.float32)
  g_ref[...] = h * dinv_ref[...]


def _linear_call(xp, W, dinv_col):
  return pl.pallas_call(
      _lin_body,
      grid=(N_PAD_NODES // BLK_L,),
      in_specs=[
          pl.BlockSpec((BLK_L, D_IN), lambda i: (i, 0)),
          pl.BlockSpec((D_IN, D_OUT), lambda i: (0, 0)),
          pl.BlockSpec((BLK_L, 1), lambda i: (i, 0)),
      ],
      out_specs=pl.BlockSpec((BLK_L, D_OUT), lambda i: (i, 0)),
      out_shape=jax.ShapeDtypeStruct((N_PAD_NODES, D_OUT), jnp.float32),
  )(xp, W, dinv_col)


def _out_body(p_ref, dinv_ref, b_ref, o_ref):
  z = (p_ref[0] + p_ref[1]) * dinv_ref[...] + b_ref[...]
  m = jnp.max(z, axis=1, keepdims=True)
  zs = z - m
  o_ref[...] = zs - jnp.log(jnp.sum(jnp.exp(zs), axis=1, keepdims=True))


def _final_call(acc_part, dinv_col, b2):
  return pl.pallas_call(
      _out_body,
      grid=(N_NODES // BLK,),
      in_specs=[
          pl.BlockSpec((NCORES, BLK, D_OUT), lambda i: (0, i, 0)),
          pl.BlockSpec((BLK, 1), lambda i: (i, 0)),
          pl.BlockSpec((1, D_OUT), lambda i: (0, 0)),
      ],
      out_specs=pl.BlockSpec((BLK, D_OUT), lambda i: (i, 0)),
      out_shape=jax.ShapeDtypeStruct((N_NODES, D_OUT), jnp.float32),
  )(acc_part, dinv_col, b2)


def kernel(x, edge_index, W, b):
  n_edges = edge_index.shape[1]
  per_round = CHUNK * NWORKERS
  cpt = -(-n_edges // per_round)
  cpt = -(-cpt // 8) * 8
  e_pad = cpt * per_round
  pad = e_pad - n_edges

  src = jnp.concatenate(
      [edge_index[0], jnp.zeros((pad,), jnp.int32)]).reshape(-1, CHUNK)
  pad_dst = N_NODES + (jnp.arange(pad, dtype=jnp.int32)
                       % (N_PAD_NODES - N_NODES))
  dst = jnp.concatenate([edge_index[1], pad_dst]).reshape(-1, CHUNK)

  dinv2d = _hist_call(dst)
  dinv_col = dinv2d.reshape(HGRID * 128, 1)[:N_PAD_NODES]
  xp = jnp.concatenate(
      [x, jnp.zeros((N_PAD_NODES - N_NODES, D_IN), jnp.float32)])
  g = _linear_call(xp, W, dinv_col)
  acc_part = _msg_call(cpt)(g, src, dst)
  return _final_call(acc_part, dinv_col[:N_NODES], b.reshape(1, D_OUT))


# speedup vs baseline: 1.2217x; 1.0652x over previous
import functools

import jax
import jax.numpy as jnp
from jax import lax
from jax.experimental import pallas as pl
from jax.experimental.pallas import tpu as pltpu
from jax.experimental.pallas import tpu_sc as plsc

N_NODES = 10000
D_IN = 128
D_OUT = 128

NCORES = 2
NSUB = 16
NWORKERS = NCORES * NSUB
CHUNK = 128
N_PAD_NODES = 10112
PAD_ROWS_PER_TILE = N_PAD_NODES // NSUB
HGRID = 80
EB = 32
BLK = 1000
BLK_L = PAD_ROWS_PER_TILE


def _sc_mesh():
  return plsc.VectorSubcoreMesh(core_axis_name="c", subcore_axis_name="s")


def _hist_body(d_ref, dinv_ref, acc_ref):
  i = pl.program_id(0)
  n = pl.num_programs(0)

  @pl.when(i == 0)
  def _():
    acc_ref[...] = jnp.zeros_like(acc_ref)

  d = d_ref[...]
  hi = lax.shift_right_logical(d, 7)
  lo = lax.bitwise_and(d, 127)
  ia = lax.broadcasted_iota(jnp.int32, (HGRID, 1), 0)
  ib = lax.broadcasted_iota(jnp.int32, (128, 1), 0)
  acc = jnp.zeros((HGRID, 128), jnp.float32)
  for r in range(EB):
    oh_hi = (hi[r:r + 1, :] == ia).astype(jnp.float32)
    oh_lo = (lo[r:r + 1, :] == ib).astype(jnp.float32)
    acc = acc + lax.dot_general(oh_hi, oh_lo, (((1,), (1,)), ((), ())),
                                preferred_element_type=jnp.float32)
  acc_ref[...] += acc

  @pl.when(i == n - 1)
  def _():
    dinv_ref[...] = lax.rsqrt(acc_ref[...] + 1.0)


def _hist_call(dst2d):
  n_rows = dst2d.shape[0]
  return pl.pallas_call(
      _hist_body,
      grid=(n_rows // EB,),
      in_specs=[pl.BlockSpec((EB, 128), lambda i: (i, 0))],
      out_specs=pl.BlockSpec((HGRID, 128), lambda i: (0, 0)),
      out_shape=jax.ShapeDtypeStruct((HGRID, 128), jnp.float32),
      scratch_shapes=[pltpu.VMEM((HGRID, 128), jnp.float32)],
  )(dst2d)


@functools.lru_cache(maxsize=None)
def _msg_call(cpt):
  @functools.partial(
      pl.kernel,
      out_type=jax.ShapeDtypeStruct((NCORES, N_PAD_NODES, D_OUT), jnp.float32),
      mesh=_sc_mesh(),
      scratch_types=[
          pltpu.VMEM(((2 * cpt) // 5, CHUNK), jnp.int32),
          pltpu.VMEM(((2 * cpt) // 5, CHUNK), jnp.int32),
          pltpu.VMEM((CHUNK, D_OUT), jnp.float32),
          pltpu.VMEM((CHUNK, D_OUT), jnp.float32),
          pltpu.VMEM_SHARED((N_PAD_NODES, D_OUT), jnp.float32),
          pltpu.SemaphoreType.DMA,
          pltpu.SemaphoreType.DMA,
      ],
  )
  def msg_kernel(g_hbm, src_hbm, dst_hbm, out_hbm, src_v, dst_v, rows_v,
                 rows_b, acc_sh, sem, sem_b):
    c = lax.axis_index("c")
    s = lax.axis_index("s")
    wid = s * NCORES + c

    row0 = s * PAD_ROWS_PER_TILE

    @pl.when(c == 0)
    def _():
      for off in range(0, PAD_ROWS_PER_TILE, CHUNK):
        n = min(CHUNK, PAD_ROWS_PER_TILE - off)
        pltpu.sync_copy(g_hbm.at[pl.ds(row0 + off, n)], rows_v.at[pl.ds(0, n)])
        pltpu.sync_copy(rows_v.at[pl.ds(0, n)],
                        acc_sh.at[pl.ds(row0 + off, n)])

    @pl.when(c != 0)
    def _():
      def fillz(i, _):
        for k in range(8):
          rows_v[i, pl.ds(k * 16, 16)] = jnp.zeros((16,), jnp.float32)
        return 0

      lax.fori_loop(0, CHUNK, fillz, 0)
      for off in range(0, PAD_ROWS_PER_TILE, CHUNK):
        n = min(CHUNK, PAD_ROWS_PER_TILE - off)
        pltpu.sync_copy(rows_v.at[pl.ds(0, n)],
                        acc_sh.at[pl.ds(row0 + off, n)])

    plsc.subcore_barrier()

    ph = (2 * cpt) // 5

    def edge_phase(pbase):
      pltpu.sync_copy(src_hbm.at[pl.ds(pbase, ph)], src_v)
      pltpu.sync_copy(dst_hbm.at[pl.ds(pbase, ph)], dst_v)
      pltpu.async_copy(g_hbm.at[src_v.at[0]], rows_v, sem)

      def pair(t, _):
        c0 = 2 * t
        pltpu.make_async_copy(g_hbm.at[src_v.at[c0]], rows_v, sem).wait()
        pltpu.async_copy(g_hbm.at[src_v.at[c0 + 1]], rows_b, sem_b)
        pltpu.sync_copy(rows_v, acc_sh.at[dst_v.at[c0]], add=True)
        pltpu.make_async_copy(g_hbm.at[src_v.at[c0 + 1]], rows_b,
                              sem_b).wait()
        pltpu.async_copy(g_hbm.at[src_v.at[c0 + 2]], rows_v, sem)
        pltpu.sync_copy(rows_b, acc_sh.at[dst_v.at[c0 + 1]], add=True)
        return 0

      lax.fori_loop(0, ph // 2 - 1, pair, 0)
      pltpu.make_async_copy(g_hbm.at[src_v.at[ph - 2]], rows_v, sem).wait()
      pltpu.async_copy(g_hbm.at[src_v.at[ph - 1]], rows_b, sem_b)
      pltpu.sync_copy(rows_v, acc_sh.at[dst_v.at[ph - 2]], add=True)
      pltpu.make_async_copy(g_hbm.at[src_v.at[ph - 1]], rows_b, sem_b).wait()
      pltpu.sync_copy(rows_b, acc_sh.at[dst_v.at[ph - 1]], add=True)

    @pl.when(c == 0)
    def _():
      for phase in range(4):
        edge_phase(s * 4 * ph + phase * ph)

    @pl.when(c != 0)
    def _():
      edge_phase(NSUB * 4 * ph + s * ph)

    plsc.subcore_barrier()

    for off in range(0, PAD_ROWS_PER_TILE, CHUNK):
      n = min(CHUNK, PAD_ROWS_PER_TILE - off)
      pltpu.sync_copy(acc_sh.at[pl.ds(row0 + off, n)], rows_v.at[pl.ds(0, n)])
      pltpu.sync_copy(rows_v.at[pl.ds(0, n)],
                      out_hbm.at[c, pl.ds(row0 + off, n)])

  return msg_kernel


def _lin_body(x_ref, w_ref, dinv_ref, g_ref):
  h = jnp.dot(x_ref[...], w_ref[...], preferred_element_type=jnp.float32)
  g_ref[...] = h * dinv_ref[...]


def _linear_call(xp, W, dinv_col):
  return pl.pallas_call(
      _lin_body,
      grid=(N_PAD_NODES // BLK_L,),
      in_specs=[
          pl.BlockSpec((BLK_L, D_IN), lambda i: (i, 0)),
          pl.BlockSpec((D_IN, D_OUT), lambda i: (0, 0)),
          pl.BlockSpec((BLK_L, 1), lambda i: (i, 0)),
      ],
      out_specs=pl.BlockSpec((BLK_L, D_OUT), lambda i: (i, 0)),
      out_shape=jax.ShapeDtypeStruct((N_PAD_NODES, D_OUT), jnp.float32),
  )(xp, W, dinv_col)


def _out_body(p_ref, dinv_ref, b_ref, o_ref):
  z = (p_ref[0] + p_ref[1]) * dinv_ref[...] + b_ref[...]
  m = jnp.max(z, axis=1, keepdims=True)
  zs = z - m
  o_ref[...] = zs - jnp.log(jnp.sum(jnp.exp(zs), axis=1, keepdims=True))


def _final_call(acc_part, dinv_col, b2):
  return pl.pallas_call(
      _out_body,
      grid=(N_NODES // BLK,),
      in_specs=[
          pl.BlockSpec((NCORES, BLK, D_OUT), lambda i: (0, i, 0)),
          pl.BlockSpec((BLK, 1), lambda i: (i, 0)),
          pl.BlockSpec((1, D_OUT), lambda i: (0, 0)),
      ],
      out_specs=pl.BlockSpec((BLK, D_OUT), lambda i: (i, 0)),
      out_shape=jax.ShapeDtypeStruct((N_NODES, D_OUT), jnp.float32),
  )(acc_part, dinv_col, b2)


def kernel(x, edge_index, W, b):
  n_edges = edge_index.shape[1]
  per_round = CHUNK * NWORKERS
  cpt = -(-n_edges // per_round)
  cpt = -(-cpt // 8) * 8
  e_pad = cpt * per_round
  pad = e_pad - n_edges

  src = jnp.concatenate(
      [edge_index[0], jnp.zeros((pad,), jnp.int32)]).reshape(-1, CHUNK)
  pad_dst = N_NODES + (jnp.arange(pad, dtype=jnp.int32)
                       % (N_PAD_NODES - N_NODES))
  dst = jnp.concatenate([edge_index[1], pad_dst]).reshape(-1, CHUNK)

  dinv2d = _hist_call(dst)
  dinv_col = dinv2d.reshape(HGRID * 128, 1)[:N_PAD_NODES]
  xp = jnp.concatenate(
      [x, jnp.zeros((N_PAD_NODES - N_NODES, D_IN), jnp.float32)])
  g = _linear_call(xp, W, dinv_col)
  acc_part = _msg_call(cpt)(g, src, dst)
  return _final_call(acc_part, dinv_col[:N_NODES], b.reshape(1, D_OUT))
